# baseline (speedup 1.0000x reference)
import functools

import jax
import jax.numpy as jnp
from jax import lax
from jax.experimental import pallas as pl
from jax.experimental.pallas import tpu as pltpu
from jax.experimental.pallas import tpu_sc as plsc

N = 10000
E = 320000
D = 128
BN_EPS = 1e-5

NC = 2
NS = 16
NW = NC * NS

CHUNK = 128
TPT = 80
E_PAD = NW * TPT * CHUNK
NBUF = 2
DRING = 6
ACC_ROWS = N + 8

RPT = 624
RPT_LAST = N - 15 * RPT


def _seg_sum_body(h_hbm, srcb_hbm, dstb_hbm, zeros_hbm, out_hbm,
                  sidx, didx, rows, acc, sem_i, sem_g, sem_s):
    cid = lax.axis_index("c")
    sid = lax.axis_index("s")
    wid = cid * NS + sid

    icp_s = pltpu.async_copy(srcb_hbm.at[wid], sidx, sem_i)
    icp_d = pltpu.async_copy(dstb_hbm.at[wid, pl.ds(0, DRING)], didx, sem_i)

    base = sid * RPT

    @pl.when(sid < NS - 1)
    def _():
        pltpu.sync_copy(zeros_hbm.at[pl.ds(base, RPT)],
                        acc.at[pl.ds(base, RPT)])

    @pl.when(sid == NS - 1)
    def _():
        pltpu.sync_copy(zeros_hbm.at[pl.ds(base, RPT_LAST)],
                        acc.at[pl.ds(base, RPT_LAST)])

    plsc.subcore_barrier()
    icp_s.wait()
    icp_d.wait()

    pltpu.async_copy(h_hbm.at[sidx.at[0]], rows.at[0], sem_g)

    def step(j, carry):
        b = lax.rem(j, NBUF)
        jd = lax.rem(j, DRING)
        pltpu.make_async_copy(h_hbm.at[sidx.at[j]], rows.at[b], sem_g).wait()
        pltpu.async_copy(rows.at[b], acc.at[didx.at[jd]], sem_s, add=True)

        @pl.when(j >= 1)
        def _():
            bp = lax.rem(j - 1, NBUF)
            jp = lax.rem(j - 1, DRING)
            pltpu.make_async_copy(rows.at[bp],
                                  acc.at[didx.at[jp]], sem_s).wait()

        @pl.when(j + DRING - 1 < TPT)
        def _():
            jn = j + DRING - 1
            pltpu.async_copy(dstb_hbm.at[wid, jn], didx.at[lax.rem(jn, DRING)],
                             sem_i)

        @pl.when(jnp.logical_and(j >= 1, j + DRING - 2 < TPT))
        def _():
            jn = j + DRING - 2
            pltpu.make_async_copy(dstb_hbm.at[wid, jn],
                                  didx.at[lax.rem(jn, DRING)], sem_i).wait()

        @pl.when(j + 1 < TPT)
        def _():
            bn = lax.rem(j + 1, NBUF)
            pltpu.async_copy(h_hbm.at[sidx.at[j + 1]], rows.at[bn], sem_g)

        return carry

    lax.fori_loop(0, TPT, step, 0)

    bl = (TPT - 1) % NBUF
    jl = (TPT - 1) % DRING
    pltpu.make_async_copy(rows.at[bl], acc.at[didx.at[jl]], sem_s).wait()
    plsc.subcore_barrier()

    @pl.when(sid < NS - 1)
    def _():
        pltpu.sync_copy(acc.at[pl.ds(base, RPT)],
                        out_hbm.at[pl.ds(cid * N + base, RPT)])

    @pl.when(sid == NS - 1)
    def _():
        pltpu.sync_copy(acc.at[pl.ds(base, RPT_LAST)],
                        out_hbm.at[pl.ds(cid * N + base, RPT_LAST)])


_seg_sum = pl.kernel(
    _seg_sum_body,
    out_type=jax.ShapeDtypeStruct((NC * N, D), jnp.float32),
    mesh=plsc.VectorSubcoreMesh(core_axis_name="c", subcore_axis_name="s"),
    scratch_types=[
        pltpu.VMEM((TPT, CHUNK), jnp.int32),
        pltpu.VMEM((DRING, CHUNK), jnp.int32),
        pltpu.VMEM((NBUF, CHUNK, D), jnp.float32),
        pltpu.VMEM_SHARED((ACC_ROWS, D), jnp.float32),
        pltpu.SemaphoreType.DMA,
        pltpu.SemaphoreType.DMA,
        pltpu.SemaphoreType.DMA,
    ],
)


def _bn(a, g, b):
    m = jnp.mean(a, axis=0)
    v = jnp.mean((a - m) * (a - m), axis=0)
    return (a - m) * lax.rsqrt(v + BN_EPS) * g + b


def _dense_body(h_ref, part_ref, w1_ref, b1_ref, gi_ref, bi_ref,
                w2_ref, b2_ref, go_ref, bo_ref, o_ref, *, relu_out):
    s = h_ref[...] + part_ref[:N] + part_ref[N:]
    a = jnp.dot(s, w1_ref[...], preferred_element_type=jnp.float32)
    a = a + b1_ref[...]
    a = jnp.maximum(_bn(a, gi_ref[...], bi_ref[...]), 0.0)
    o = jnp.dot(a, w2_ref[...], preferred_element_type=jnp.float32)
    o = o + b2_ref[...]
    o = _bn(o, go_ref[...], bo_ref[...])
    if relu_out:
        o = jnp.maximum(o, 0.0)
    o_ref[...] = o


def _dense(h, part, w1, b1, gi, bi, w2, b2, go, bo, relu_out):
    return pl.pallas_call(
        functools.partial(_dense_body, relu_out=relu_out),
        out_shape=jax.ShapeDtypeStruct((N, D), jnp.float32),
    )(h, part, w1, b1, gi, bi, w2, b2, go, bo)


def kernel(x, edge_index, batch,
           w1_0, b1_0, gi_0, bi_0, w2_0, b2_0, go_0, bo_0,
           w1_1, b1_1, gi_1, bi_1, w2_1, b2_1, go_1, bo_1,
           w1_2, b1_2, gi_2, bi_2, w2_2, b2_2, go_2, bo_2):
    src = edge_index[0]
    dst = edge_index[1]
    pad = E_PAD - E
    srcb = jnp.concatenate([src, jnp.zeros((pad,), jnp.int32)])
    srcb = srcb.reshape(NW, TPT, CHUNK)
    dstb = jnp.concatenate([dst, jnp.full((pad,), N, jnp.int32)])
    dstb = dstb.reshape(NW, TPT, CHUNK)
    zeros = jnp.zeros((N, D), jnp.float32)

    params = [
        (w1_0, b1_0, gi_0, bi_0, w2_0, b2_0, go_0, bo_0),
        (w1_1, b1_1, gi_1, bi_1, w2_1, b2_1, go_1, bo_1),
        (w1_2, b1_2, gi_2, bi_2, w2_2, b2_2, go_2, bo_2),
    ]

    h = x
    for l in range(3):
        part = _seg_sum(h, srcb, dstb, zeros)
        h = _dense(h, part, *params[l], relu_out=(l < 2))
    return h
